# baseline (speedup 1.0000x reference)
import functools

import jax
import jax.numpy as jnp
from jax import lax
from jax.experimental import pallas as pl
from jax.experimental.pallas import tpu as pltpu
from jax.experimental.pallas import tpu_sc as plsc

N = 10000
D = 128
DW = D // 2
B = 320000
NW = 32
PER_W = B // NW
CHUNK = 80
N_CHUNKS = PER_W // CHUNK
GROUPS = CHUNK // 16
N_PAIRS = (N_CHUNKS + 1) // 2
ROWS_PER_SUB = N // 16
CBLK = 25
N_CBLK = ROWS_PER_SUB // CBLK

_mesh = plsc.VectorSubcoreMesh(core_axis_name="c", subcore_axis_name="s")


@functools.partial(
    pl.kernel,
    out_type=jax.ShapeDtypeStruct((B,), jnp.float32),
    mesh=_mesh,
    compiler_params=pltpu.CompilerParams(needs_layout_passes=False,
                                         use_tc_tiling_on_sc=False),
    scratch_types=[
        pltpu.VMEM_SHARED((N, DW), jnp.int32),
        pltpu.VMEM((CBLK, D), jnp.float32),
        pltpu.VMEM((CBLK, DW), jnp.int32),
        pltpu.VMEM((PER_W,), jnp.int32),
        pltpu.VMEM((PER_W,), jnp.int32),
        pltpu.VMEM((CHUNK, DW), jnp.int32),
        pltpu.VMEM((CHUNK, DW), jnp.int32),
        pltpu.VMEM((CHUNK, DW), jnp.int32),
        pltpu.VMEM((CHUNK, DW), jnp.int32),
        pltpu.VMEM((CHUNK, DW), jnp.int32),
        pltpu.VMEM((CHUNK, DW), jnp.int32),
        pltpu.VMEM((CHUNK, DW), jnp.int32),
        pltpu.VMEM((CHUNK, DW), jnp.int32),
        pltpu.VMEM((PER_W,), jnp.float32),
        pltpu.SemaphoreType.DMA,
        pltpu.SemaphoreType.DMA,
        pltpu.SemaphoreType.DMA,
        pltpu.SemaphoreType.DMA,
        pltpu.SemaphoreType.DMA,
        pltpu.SemaphoreType.DMA,
        pltpu.SemaphoreType.DMA,
        pltpu.SemaphoreType.DMA,
    ],
)
def _edge_dot(x_hbm, eidx_hbm, out_hbm,
              x_sp, cf32, ci32, sidx, didx,
              srows0, srows1, srows2, srows3,
              drows0, drows1, drows2, drows3, outv,
              ssem0, ssem1, ssem2, ssem3,
              dsem0, dsem1, dsem2, dsem3):
    sub = lax.axis_index("s")
    wid = sub * 2 + lax.axis_index("c")
    base = wid * PER_W

    row0 = sub * ROWS_PER_SUB
    for blk in range(N_CBLK):
        r0 = row0 + blk * CBLK
        pltpu.sync_copy(x_hbm.at[pl.ds(r0, CBLK)], cf32)

        def pack_row(r, carry):
            for k in range(DW // 16):
                a = cf32[r, pl.ds(k * 32, 16)]
                b = cf32[r, pl.ds(k * 32 + 16, 16)]
                packed = plsc.pack(a, b, format=plsc.PackFormat.INTERLEAVED)
                ci32[r, pl.ds(k * 16, 16)] = plsc.bitcast(packed, jnp.int32)
            return carry

        lax.fori_loop(0, CBLK, pack_row, 0)
        pltpu.sync_copy(ci32, x_sp.at[pl.ds(r0, CBLK)])

    pltpu.sync_copy(eidx_hbm.at[0, pl.ds(base, PER_W)], sidx)
    pltpu.sync_copy(eidx_hbm.at[1, pl.ds(base, PER_W)], didx)
    plsc.subcore_barrier()

    sbufs = (srows0, srows1, srows2, srows3)
    dbufs = (drows0, drows1, drows2, drows3)
    ssems = (ssem0, ssem1, ssem2, ssem3)
    dsems = (dsem0, dsem1, dsem2, dsem3)

    def issue(c, b):
        pltpu.async_copy(x_sp.at[sidx.at[pl.ds(c * CHUNK, CHUNK)]],
                         sbufs[b], ssems[b])
        pltpu.async_copy(x_sp.at[didx.at[pl.ds(c * CHUNK, CHUNK)]],
                         dbufs[b], dsems[b])

    def wait(b):
        pltpu.make_async_copy(x_hbm.at[pl.ds(0, CHUNK // 2)], sbufs[b],
                              ssems[b]).wait()
        pltpu.make_async_copy(x_hbm.at[pl.ds(0, CHUNK // 2)], dbufs[b],
                              dsems[b]).wait()

    issue(0, 0)
    issue(1, 1)
    issue(2, 2)
    issue(3, 3)

    lanes = lax.iota(jnp.int32, 16)

    def pair_body(i, carry):
        for b in range(4):
            c = 4 * i + b
            srows = sbufs[b]
            drows = dbufs[b]

            @pl.when(c < N_CHUNKS)
            def _():
                wait(b)
                for g in range(GROUPS):

                    def ebody(j, res):
                        e = g * 16 + j
                        sv = plsc.bitcast(srows[e, pl.ds(0, 16)],
                                          jnp.bfloat16)
                        tv = plsc.bitcast(drows[e, pl.ds(0, 16)],
                                          jnp.bfloat16)
                        accp = sv * tv
                        for k in range(1, DW // 16):
                            sv = plsc.bitcast(srows[e, pl.ds(k * 16, 16)],
                                              jnp.bfloat16)
                            tv = plsc.bitcast(drows[e, pl.ds(k * 16, 16)],
                                              jnp.bfloat16)
                            accp = accp + sv * tv
                        p0, p1 = plsc.unpack(
                            accp, format=plsc.PackFormat.INTERLEAVED)
                        dot = lax.reduce_sum(p0 + p1, axes=(0,))
                        return jnp.where(lanes == j, jnp.full((16,), dot), res)

                    res = lax.fori_loop(0, 16, ebody,
                                        jnp.zeros((16,), jnp.float32))
                    outv[pl.ds(c * CHUNK + g * 16, 16)] = res

                @pl.when(c + 4 < N_CHUNKS)
                def _():
                    issue(c + 4, b)

        return carry

    lax.fori_loop(0, (N_CHUNKS + 3) // 4, pair_body, 0)
    pltpu.sync_copy(outv, out_hbm.at[pl.ds(base, PER_W)])


def kernel(x, edge_label_index):
    return _edge_dot(x, edge_label_index)


# speedup vs baseline: 1.1985x; 1.1985x over previous
import functools

import jax
import jax.numpy as jnp
from jax import lax
from jax.experimental import pallas as pl
from jax.experimental.pallas import tpu as pltpu
from jax.experimental.pallas import tpu_sc as plsc

N = 10000
D = 128
DW = D // 2
B = 320000
NW = 32
PER_W = B // NW
CHUNK = 80
N_CHUNKS = PER_W // CHUNK
GROUPS = CHUNK // 16
N_PAIRS = (N_CHUNKS + 1) // 2
ROWS_PER_SUB = N // 16
CBLK = 125
N_CBLK = ROWS_PER_SUB // CBLK

_mesh = plsc.VectorSubcoreMesh(core_axis_name="c", subcore_axis_name="s")


@functools.partial(
    pl.kernel,
    out_type=jax.ShapeDtypeStruct((B,), jnp.float32),
    mesh=_mesh,
    compiler_params=pltpu.CompilerParams(needs_layout_passes=False,
                                         use_tc_tiling_on_sc=False),
    scratch_types=[
        pltpu.VMEM_SHARED((N, DW), jnp.int32),
        pltpu.VMEM((CBLK, D), jnp.float32),
        pltpu.VMEM((CBLK, D), jnp.float32),
        pltpu.VMEM((CBLK, DW), jnp.int32),
        pltpu.VMEM((PER_W,), jnp.int32),
        pltpu.VMEM((PER_W,), jnp.int32),
        pltpu.VMEM((CHUNK, DW), jnp.int32),
        pltpu.VMEM((CHUNK, DW), jnp.int32),
        pltpu.VMEM((CHUNK, DW), jnp.int32),
        pltpu.VMEM((CHUNK, DW), jnp.int32),
        pltpu.VMEM((PER_W,), jnp.float32),
        pltpu.SemaphoreType.DMA,
        pltpu.SemaphoreType.DMA,
        pltpu.SemaphoreType.DMA,
        pltpu.SemaphoreType.DMA,
        pltpu.SemaphoreType.DMA,
        pltpu.SemaphoreType.DMA,
    ],
)
def _edge_dot(x_hbm, eidx_hbm, out_hbm,
              x_sp, cf32a, cf32b, ci32, sidx, didx,
              srows0, srows1, drows0, drows1, outv,
              ssem0, ssem1, dsem0, dsem1, csem0, csem1):
    sub = lax.axis_index("s")
    wid = sub * 2 + lax.axis_index("c")
    base = wid * PER_W

    row0 = sub * ROWS_PER_SUB
    cbufs = (cf32a, cf32b)
    csems = (csem0, csem1)
    pltpu.async_copy(x_hbm.at[pl.ds(row0, CBLK)], cf32a, csem0)
    for blk in range(N_CBLK):
        r0 = row0 + blk * CBLK
        buf = cbufs[blk % 2]
        pltpu.make_async_copy(x_hbm.at[pl.ds(0, CBLK)], buf,
                              csems[blk % 2]).wait()
        if blk + 1 < N_CBLK:
            pltpu.async_copy(x_hbm.at[pl.ds(r0 + CBLK, CBLK)],
                             cbufs[(blk + 1) % 2], csems[(blk + 1) % 2])

        def pack_row(r, carry):
            for k in range(DW // 16):
                a = buf[r, pl.ds(k * 32, 16)]
                b = buf[r, pl.ds(k * 32 + 16, 16)]
                packed = plsc.pack(a, b, format=plsc.PackFormat.INTERLEAVED)
                ci32[r, pl.ds(k * 16, 16)] = plsc.bitcast(packed, jnp.int32)
            return carry

        lax.fori_loop(0, CBLK, pack_row, 0)
        pltpu.sync_copy(ci32, x_sp.at[pl.ds(r0, CBLK)])

    pltpu.sync_copy(eidx_hbm.at[0, pl.ds(base, PER_W)], sidx)
    pltpu.sync_copy(eidx_hbm.at[1, pl.ds(base, PER_W)], didx)
    plsc.subcore_barrier()

    sbufs = (srows0, srows1)
    dbufs = (drows0, drows1)
    ssems = (ssem0, ssem1)
    dsems = (dsem0, dsem1)

    def issue(c, b):
        pltpu.async_copy(x_sp.at[sidx.at[pl.ds(c * CHUNK, CHUNK)]],
                         sbufs[b], ssems[b])
        pltpu.async_copy(x_sp.at[didx.at[pl.ds(c * CHUNK, CHUNK)]],
                         dbufs[b], dsems[b])

    def wait(b):
        pltpu.make_async_copy(x_hbm.at[pl.ds(0, CHUNK // 2)], sbufs[b],
                              ssems[b]).wait()
        pltpu.make_async_copy(x_hbm.at[pl.ds(0, CHUNK // 2)], dbufs[b],
                              dsems[b]).wait()

    issue(0, 0)
    issue(1, 1)

    lanes = lax.iota(jnp.int32, 16)

    def pair_body(i, carry):
        for b in range(2):
            c = 2 * i + b
            srows = sbufs[b]
            drows = dbufs[b]

            @pl.when(c < N_CHUNKS)
            def _():
                wait(b)
                for g in range(GROUPS):

                    def ebody(j, res):
                        e = g * 16 + j
                        sv = plsc.bitcast(srows[e, pl.ds(0, 16)],
                                          jnp.bfloat16)
                        tv = plsc.bitcast(drows[e, pl.ds(0, 16)],
                                          jnp.bfloat16)
                        accp = sv * tv
                        for k in range(1, DW // 16):
                            sv = plsc.bitcast(srows[e, pl.ds(k * 16, 16)],
                                              jnp.bfloat16)
                            tv = plsc.bitcast(drows[e, pl.ds(k * 16, 16)],
                                              jnp.bfloat16)
                            accp = accp + sv * tv
                        p0, p1 = plsc.unpack(
                            accp, format=plsc.PackFormat.INTERLEAVED)
                        dot = lax.reduce_sum(p0 + p1, axes=(0,))
                        return jnp.where(lanes == j, jnp.full((16,), dot), res)

                    res = lax.fori_loop(0, 16, ebody,
                                        jnp.zeros((16,), jnp.float32))
                    outv[pl.ds(c * CHUNK + g * 16, 16)] = res

                @pl.when(c + 2 < N_CHUNKS)
                def _():
                    issue(c + 2, b)

        return carry

    lax.fori_loop(0, N_PAIRS, pair_body, 0)
    pltpu.sync_copy(outv, out_hbm.at[pl.ds(base, PER_W)])


def kernel(x, edge_label_index):
    return _edge_dot(x, edge_label_index)
